# baseline (speedup 1.0000x reference)
import functools
import math

import jax
import jax.numpy as jnp
from jax.experimental import pallas as pl
from jax.experimental.pallas import tpu as pltpu

_N_BOXES = 20000
_NUM_CLASSES = 80
_NUM_PREDS = 100
_IOU_THR = 0.5
_SCORE_THR = 0.3
_MAX_EDGE = 1024
_SCALE_CLAMP = math.log(1000.0 / 16)

_ROWS = 157
_LANES = 128
_NPAD = _ROWS * _LANES


def _split_cols(ref):
    v = jnp.concatenate(
        [ref[...], jnp.zeros((_NPAD - _N_BOXES, 4), jnp.float32)], axis=0)
    v = v.reshape(_ROWS, _LANES, 4)
    v = jnp.transpose(v, (0, 2, 1))
    return (v[:, 0, :], v[:, 1, :], v[:, 2, :], v[:, 3, :])


def _body(l_hbm, anchors_ref, deltas_ref, pred_ref,
          l_vmem, x1c_ref, y1c_ref, x2c_ref, y2c_ref, w_ref, dma_sem):
    f32 = jnp.float32

    cp = pltpu.make_async_copy(l_hbm, l_vmem, dma_sem)
    cp.start()

    ax1, ay1, ax2, ay2 = _split_cols(anchors_ref)
    dxv, dyv, dwv, dhv = _split_cols(deltas_ref)
    widths = ax2 - ax1
    heights = ay2 - ay1
    ctr_x = (ax1 + ax2) * 0.5
    ctr_y = (ay1 + ay2) * 0.5
    dw = jnp.minimum(dwv, _SCALE_CLAMP)
    dh = jnp.minimum(dhv, _SCALE_CLAMP)
    pred_ctr_x = dxv * widths + ctr_x
    pred_ctr_y = dyv * heights + ctr_y
    pred_w = jnp.exp(dw) * widths
    pred_h = jnp.exp(dh) * heights
    hi = f32(_MAX_EDGE - 1.0)
    x1 = jnp.clip(pred_ctr_x - 0.5 * pred_w, 0.0, hi)
    y1 = jnp.clip(pred_ctr_y - 0.5 * pred_h, 0.0, hi)
    x2 = jnp.clip(pred_ctr_x + 0.5 * pred_w, 0.0, hi)
    y2 = jnp.clip(pred_ctr_y + 0.5 * pred_h, 0.0, hi)

    cp.wait()
    l = jnp.concatenate(
        [l_vmem[...],
         jnp.full((_NPAD - _N_BOXES, _NUM_CLASSES), -100.0, jnp.float32)],
        axis=0).reshape(_ROWS, _LANES, _NUM_CLASSES)
    p = jax.nn.sigmoid(l)
    m = jnp.max(p, axis=2)
    cat = jnp.argmax(p, axis=2)
    catf = cat.astype(f32)
    s0 = jnp.where(m >= _SCORE_THR, m, -1.0)

    off = catf * f32(_MAX_EDGE)
    x1c = x1 + off
    y1c = y1 + off
    x2c = x2 + off
    y2c = y2 + off
    area = (x2c - x1c) * (y2c - y1c)

    x1c_ref[...] = x1c
    y1c_ref[...] = y1c
    x2c_ref[...] = x2c
    y2c_ref[...] = y2c
    w_ref[:, 0, :] = x1c
    w_ref[:, 1, :] = y1c
    w_ref[:, 2, :] = x2c
    w_ref[:, 3, :] = y2c
    w_ref[:, 4, :] = catf
    w_ref[:, 5, :] = area
    w_ref[:, 6, :] = area
    w_ref[:, 7, :] = area

    flat_idx = (jax.lax.broadcasted_iota(jnp.int32, (_ROWS, _LANES), 0) * _LANES
                + jax.lax.broadcasted_iota(jnp.int32, (_ROWS, _LANES), 1))
    lane8 = jax.lax.broadcasted_iota(jnp.int32, (1, 8, _LANES), 2)
    lane = jax.lax.broadcasted_iota(jnp.int32, (1, _LANES), 1)

    def nms_step(i, carry):
        s, mx = carry
        j = jnp.min(jnp.where(s == mx, flat_idx, jnp.int32(2**30)))
        r = j // _LANES
        c = j - r * _LANES

        wrow = w_ref[pl.ds(r, 1)]
        w8 = jnp.sum(jnp.where(lane8 == c, wrow, 0.0), axis=2)

        def bc(k):
            return jax.lax.broadcast_in_dim(w8[:, k], (_ROWS, _LANES), (1,))

        wx1 = bc(0)
        wy1 = bc(1)
        wx2 = bc(2)
        wy2 = bc(3)
        warea = bc(5)

        xx1 = jnp.maximum(wx1, x1c_ref[...])
        yy1 = jnp.maximum(wy1, y1c_ref[...])
        xx2 = jnp.minimum(wx2, x2c_ref[...])
        yy2 = jnp.minimum(wy2, y2c_ref[...])
        inter = jnp.maximum(xx2 - xx1, 0.0) * jnp.maximum(yy2 - yy1, 0.0)
        iou = inter / (warea + area - inter + 1e-9)
        suppress = (iou > _IOU_THR) | (flat_idx == j)
        s_new = jnp.where(suppress, -1.0, s)
        mx_new = jnp.max(s_new, axis=(0, 1), keepdims=True)

        def bl(k):
            return jax.lax.broadcast_in_dim(w8[:, k], (1, _LANES), (0,))

        wcat_l = bl(4)
        woff_l = wcat_l * f32(_MAX_EDGE)
        mx_l = jax.lax.broadcast_in_dim(mx[0], (1, _LANES), (0,))
        row = jnp.where(lane == 0, wcat_l,
              jnp.where(lane == 1, mx_l,
              jnp.where(lane == 2, bl(0) - woff_l,
              jnp.where(lane == 3, bl(1) - woff_l,
              jnp.where(lane == 4, bl(2) - woff_l,
              jnp.where(lane == 5, bl(3) - woff_l, -1.0))))))
        row = jnp.where(mx > 0.0, row, -1.0)
        pred_ref[pl.ds(i, 1), :] = row
        return (s_new, mx_new)

    m0 = jnp.max(s0, axis=(0, 1), keepdims=True)
    jax.lax.fori_loop(0, _NUM_PREDS, nms_step, (s0, m0))


@jax.jit
def kernel(anchors, deltas, logits):
    f32 = jnp.float32

    pred = pl.pallas_call(
        _body,
        out_shape=jax.ShapeDtypeStruct((_NUM_PREDS, _LANES), f32),
        in_specs=[
            pl.BlockSpec(memory_space=pltpu.HBM),
            pl.BlockSpec(memory_space=pltpu.VMEM),
            pl.BlockSpec(memory_space=pltpu.VMEM),
        ],
        out_specs=pl.BlockSpec(memory_space=pltpu.VMEM),
        scratch_shapes=[
            pltpu.VMEM((_N_BOXES, _NUM_CLASSES), f32),
            pltpu.VMEM((_ROWS, _LANES), f32),
            pltpu.VMEM((_ROWS, _LANES), f32),
            pltpu.VMEM((_ROWS, _LANES), f32),
            pltpu.VMEM((_ROWS, _LANES), f32),
            pltpu.VMEM((_ROWS, 8, _LANES), f32),
            pltpu.SemaphoreType.DMA,
        ],
    )(logits, anchors, deltas)

    return pred[:, :6]


# speedup vs baseline: 1.3748x; 1.3748x over previous
import functools
import math

import jax
import jax.numpy as jnp
from jax.experimental import pallas as pl
from jax.experimental.pallas import tpu as pltpu

_N_BOXES = 20000
_NUM_CLASSES = 80
_NUM_PREDS = 100
_IOU_THR = 0.5
_SCORE_THR = 0.3
_MAX_EDGE = 1024
_SCALE_CLAMP = math.log(1000.0 / 16)

_ROWS = 157
_LANES = 128
_NPAD = _ROWS * _LANES


def _body(l_ref, ax1_ref, ay1_ref, ax2_ref, ay2_ref,
          dx_ref, dy_ref, dw_ref, dh_ref, pred_ref,
          x1c_ref, y1c_ref, x2c_ref, y2c_ref, w_ref):
    f32 = jnp.float32

    ax1, ay1, ax2, ay2 = ax1_ref[...], ay1_ref[...], ax2_ref[...], ay2_ref[...]
    dxv, dyv, dwv, dhv = dx_ref[...], dy_ref[...], dw_ref[...], dh_ref[...]
    widths = ax2 - ax1
    heights = ay2 - ay1
    ctr_x = (ax1 + ax2) * 0.5
    ctr_y = (ay1 + ay2) * 0.5
    dw = jnp.minimum(dwv, _SCALE_CLAMP)
    dh = jnp.minimum(dhv, _SCALE_CLAMP)
    pred_ctr_x = dxv * widths + ctr_x
    pred_ctr_y = dyv * heights + ctr_y
    pred_w = jnp.exp(dw) * widths
    pred_h = jnp.exp(dh) * heights
    hi = f32(_MAX_EDGE - 1.0)
    x1 = jnp.clip(pred_ctr_x - 0.5 * pred_w, 0.0, hi)
    y1 = jnp.clip(pred_ctr_y - 0.5 * pred_h, 0.0, hi)
    x2 = jnp.clip(pred_ctr_x + 0.5 * pred_w, 0.0, hi)
    y2 = jnp.clip(pred_ctr_y + 0.5 * pred_h, 0.0, hi)

    l = jnp.concatenate(
        [l_ref[...],
         jnp.full((_NPAD - _N_BOXES, _NUM_CLASSES), -100.0, jnp.float32)],
        axis=0).reshape(_ROWS, _LANES, _NUM_CLASSES)
    p = jax.nn.sigmoid(l)
    m = jnp.max(p, axis=2)
    cat = jnp.argmax(p, axis=2)
    catf = cat.astype(f32)
    s0 = jnp.where(m >= _SCORE_THR, m, -1.0)

    off = catf * f32(_MAX_EDGE)
    x1c = x1 + off
    y1c = y1 + off
    x2c = x2 + off
    y2c = y2 + off
    area = (x2c - x1c) * (y2c - y1c)

    x1c_ref[...] = x1c
    y1c_ref[...] = y1c
    x2c_ref[...] = x2c
    y2c_ref[...] = y2c
    w_ref[:, 0, :] = x1c
    w_ref[:, 1, :] = y1c
    w_ref[:, 2, :] = x2c
    w_ref[:, 3, :] = y2c
    w_ref[:, 4, :] = catf
    w_ref[:, 5, :] = area
    w_ref[:, 6, :] = area
    w_ref[:, 7, :] = area

    flat_idx = (jax.lax.broadcasted_iota(jnp.int32, (_ROWS, _LANES), 0) * _LANES
                + jax.lax.broadcasted_iota(jnp.int32, (_ROWS, _LANES), 1))
    lane8 = jax.lax.broadcasted_iota(jnp.int32, (1, 8, _LANES), 2)
    lane = jax.lax.broadcasted_iota(jnp.int32, (1, _LANES), 1)

    def nms_step(i, carry):
        s, mx = carry
        j = jnp.min(jnp.where(s == mx, flat_idx, jnp.int32(2**30)))
        r = j // _LANES
        c = j - r * _LANES

        wrow = w_ref[pl.ds(r, 1)]
        w8 = jnp.sum(jnp.where(lane8 == c, wrow, 0.0), axis=2)

        def bc(k):
            return jax.lax.broadcast_in_dim(w8[:, k], (_ROWS, _LANES), (1,))

        wx1 = bc(0)
        wy1 = bc(1)
        wx2 = bc(2)
        wy2 = bc(3)
        warea = bc(5)

        xx1 = jnp.maximum(wx1, x1c_ref[...])
        yy1 = jnp.maximum(wy1, y1c_ref[...])
        xx2 = jnp.minimum(wx2, x2c_ref[...])
        yy2 = jnp.minimum(wy2, y2c_ref[...])
        inter = jnp.maximum(xx2 - xx1, 0.0) * jnp.maximum(yy2 - yy1, 0.0)
        iou = inter / (warea + area - inter + 1e-9)
        suppress = (iou > _IOU_THR) | (flat_idx == j)
        s_new = jnp.where(suppress, -1.0, s)
        mx_new = jnp.max(s_new, axis=(0, 1), keepdims=True)

        def bl(k):
            return jax.lax.broadcast_in_dim(w8[:, k], (1, _LANES), (0,))

        wcat_l = bl(4)
        woff_l = wcat_l * f32(_MAX_EDGE)
        mx_l = jax.lax.broadcast_in_dim(mx[0], (1, _LANES), (0,))
        row = jnp.where(lane == 0, wcat_l,
              jnp.where(lane == 1, mx_l,
              jnp.where(lane == 2, bl(0) - woff_l,
              jnp.where(lane == 3, bl(1) - woff_l,
              jnp.where(lane == 4, bl(2) - woff_l,
              jnp.where(lane == 5, bl(3) - woff_l, -1.0))))))
        row = jnp.where(mx > 0.0, row, -1.0)
        pred_ref[pl.ds(i, 1), :] = row
        return (s_new, mx_new)

    m0 = jnp.max(s0, axis=(0, 1), keepdims=True)
    jax.lax.fori_loop(0, _NUM_PREDS, nms_step, (s0, m0))


@jax.jit
def kernel(anchors, deltas, logits):
    f32 = jnp.float32
    pad = _NPAD - _N_BOXES

    def col(a, k):
        return jnp.pad(a[:, k], (0, pad)).reshape(_ROWS, _LANES)

    ax1, ay1, ax2, ay2 = (col(anchors, k) for k in range(4))
    dx, dy, dw, dh = (col(deltas, k) for k in range(4))

    pred = pl.pallas_call(
        _body,
        out_shape=jax.ShapeDtypeStruct((_NUM_PREDS, _LANES), f32),
        in_specs=[pl.BlockSpec(memory_space=pltpu.VMEM)] * 9,
        out_specs=pl.BlockSpec(memory_space=pltpu.VMEM),
        scratch_shapes=[
            pltpu.VMEM((_ROWS, _LANES), f32),
            pltpu.VMEM((_ROWS, _LANES), f32),
            pltpu.VMEM((_ROWS, _LANES), f32),
            pltpu.VMEM((_ROWS, _LANES), f32),
            pltpu.VMEM((_ROWS, 8, _LANES), f32),
        ],
    )(logits, ax1, ay1, ax2, ay2, dx, dy, dw, dh)

    return pred[:, :6]
